# baseline (speedup 1.0000x reference)
import functools

import jax
import jax.numpy as jnp
from jax import lax
from jax.experimental import pallas as pl
from jax.experimental.pallas import tpu as pltpu
from jax.experimental.pallas import tpu_sc as plsc

_B, _A, _C = 32, 20000, 81
_NEG_RATIO = 3
_ALPHA = 1.0
_NROW = 157
_APAD = _NROW * 128
_LAG = 8
_RB = 4 * _A // 128


def _smooth_l1(x):
    ax = jnp.abs(x)
    return jnp.where(ax < 1.0, 0.5 * x * x, ax - 0.5)


def _sortable_u32(x):
    ub = lax.bitcast_convert_type(x, jnp.uint32)
    neg = (ub >> jnp.uint32(31)) == jnp.uint32(1)
    return jnp.where(neg, ~ub, ub | jnp.uint32(0x80000000))


def _unsortable_f32(u):
    bits = jnp.where(u >= jnp.uint32(0x80000000), u ^ jnp.uint32(0x80000000), ~u)
    return lax.bitcast_convert_type(bits, jnp.float32)


def _sc_gather_body(lab_hbm, tgt_hbm, g_hbm, tgt_v, idx_v, val_v, sem):
    wid = lax.axis_index("s") * 2 + lax.axis_index("c")
    base = wid * _A
    pltpu.sync_copy(tgt_hbm.at[pl.ds(base, _A)], tgt_v)

    io16 = lax.iota(jnp.int32, 16)
    rowbase = wid * (_C * _A)

    zero16 = jnp.zeros((16,), jnp.int32)
    for c8 in range(8):
        idx_v[_NROW - 1, pl.ds(16 * c8, 16)] = zero16

    def _chunk(r, c8):
        a = r * 128 + c8 * 16
        t16 = tgt_v[pl.ds(a, 16)]
        idx_v[r, pl.ds(c8 * 16, 16)] = t16 * _A + (rowbase + a) + io16

    def _fire(r):
        pltpu.async_copy(lab_hbm.at[idx_v.at[r]], val_v.at[r], sem)

    def _drain(r):
        pltpu.make_async_copy(lab_hbm.at[idx_v.at[r]], val_v.at[r], sem).wait()

    def _row(r, carry):
        for c8 in range(8):
            _chunk(r, c8)
        _fire(r)

        @pl.when(r >= _LAG)
        def _():
            _drain(r - _LAG)

        return carry

    lax.fori_loop(0, _NROW - 1, _row, 0)

    for c8 in range(2):
        _chunk(_NROW - 1, c8)
    _fire(_NROW - 1)

    def _dr(r, carry):
        _drain(r)
        return carry

    lax.fori_loop(_NROW - 1 - _LAG, _NROW, _dr, 0)

    pltpu.sync_copy(val_v, g_hbm.at[wid])


def _sc_gather(label_flat, tgt_flat):
    mesh = plsc.VectorSubcoreMesh(core_axis_name="c", subcore_axis_name="s")
    k = functools.partial(
        pl.kernel,
        out_type=jax.ShapeDtypeStruct((_B, _NROW, 128), jnp.float32),
        mesh=mesh,
        scratch_types=[
            pltpu.VMEM((_A,), jnp.int32),
            pltpu.VMEM((_NROW, 128), jnp.int32),
            pltpu.VMEM((_NROW, 128), jnp.float32),
            pltpu.SemaphoreType.DMA,
        ],
    )(_sc_gather_body)
    return k(label_flat, tgt_flat)


def _tc_body(bi_ref, bt_ref, g_ref, tgt_ref, tgtb_ref, main_ref, bbox_ref,
             lls_ref, facc_ref):
    i = pl.program_id(0)

    @pl.when(i == 0)
    def _init():
        facc_ref[0] = 0.0
        facc_ref[1] = 0.0
        facc_ref[2] = 0.0

    tgt = tgt_ref[...].reshape(1, _A)
    pos = tgt > 0
    posf = pos.astype(jnp.float32)
    npos = jnp.sum(pos.astype(jnp.int32))
    k = jnp.minimum(_NEG_RATIO * npos, _A)

    ll = -g_ref[...].reshape(1, _APAD)[:, :_A]

    sum_pos = jnp.sum(ll * posf)

    def _fast(_):
        return jnp.sum(ll * (1.0 - posf))

    def _slow(_):
        ll2 = jnp.where(pos, 0.0, -ll)
        u = _sortable_u32(ll2)

        def bit_step(t, carry):
            prefix, kk = carry
            sh = 31 - t
            cnt0 = jnp.sum(((u >> sh) == (prefix >> sh)).astype(jnp.int32))
            take0 = kk <= cnt0
            prefix = jnp.where(take0, prefix, prefix | (jnp.uint32(1) << sh))
            kk = jnp.where(take0, kk, kk - cnt0)
            return prefix, kk

        t_u, _kk = lax.fori_loop(0, 32, bit_step, (jnp.uint32(0), k))
        lt = u < t_u
        count_lt = jnp.sum(lt.astype(jnp.int32))
        r = (k - count_lt).astype(jnp.float32)
        t_f = _unsortable_f32(t_u)
        s_lt = jnp.sum(-ll2 * jnp.where(lt, 1.0, 0.0))
        return s_lt + r * (-t_f)

    s_neg = lax.cond(k >= _A, _fast, _slow, None)
    s_neg = jnp.where(k > 0, s_neg, 0.0)

    s = _smooth_l1(bi_ref[...].reshape(_RB, 128) - bt_ref[...].reshape(_RB, 128))
    kio_l = lax.broadcasted_iota(jnp.int32, (128, 32), 0)
    kio_m = lax.broadcasted_iota(jnp.int32, (128, 32), 1)
    kmat = ((kio_l // 4) == kio_m).astype(jnp.float32)
    s4 = lax.dot_general(s, kmat, (((1,), (0,)), ((), ())),
                         preferred_element_type=jnp.float32)
    posf2 = (tgtb_ref[...].reshape(_RB, 32) > 0).astype(jnp.float32)

    facc_ref[0] += jnp.sum(s4 * posf2)
    facc_ref[1] += sum_pos + s_neg
    facc_ref[2] += npos.astype(jnp.float32)

    @pl.when(i == _B - 1)
    def _fin():
        np_t = facc_ref[2]
        bb = facc_ref[0] / np_t
        lls = facc_ref[1] / np_t
        bbox_ref[...] = jnp.reshape(bb, (1, 1))
        lls_ref[...] = jnp.reshape(lls, (1, 1))
        main_ref[...] = jnp.reshape(bb + _ALPHA * lls, (1, 1))


def kernel(bbox_input, bbox_target, label_input, label_target):
    g = _sc_gather(label_input.reshape(-1), label_target.reshape(-1))

    bi = bbox_input.reshape(_B, _RB, 128)
    bt = bbox_target.reshape(_B, _RB, 128)
    g3 = g.reshape(_B, 1, _APAD)
    tgt = label_target.reshape(_B, 1, _A)
    tgtb = label_target.reshape(_B, _RB, 32)

    out_shape = [jax.ShapeDtypeStruct((1, 1), jnp.float32)] * 3
    main, bbox, lls = pl.pallas_call(
        _tc_body,
        grid=(_B,),
        in_specs=[
            pl.BlockSpec((1, _RB, 128), lambda i: (i, 0, 0)),
            pl.BlockSpec((1, _RB, 128), lambda i: (i, 0, 0)),
            pl.BlockSpec((1, 1, _APAD), lambda i: (i, 0, 0)),
            pl.BlockSpec((1, 1, _A), lambda i: (i, 0, 0)),
            pl.BlockSpec((1, _RB, 32), lambda i: (i, 0, 0)),
        ],
        out_specs=[
            pl.BlockSpec((1, 1), lambda i: (0, 0)),
            pl.BlockSpec((1, 1), lambda i: (0, 0)),
            pl.BlockSpec((1, 1), lambda i: (0, 0)),
        ],
        out_shape=out_shape,
        scratch_shapes=[pltpu.SMEM((3,), jnp.float32)],
        compiler_params=pltpu.CompilerParams(
            dimension_semantics=("arbitrary",),
        ),
    )(bi, bt, g3, tgt, tgtb)
    return (main[0, 0], bbox[0, 0], lls[0, 0])


# speedup vs baseline: 7.3818x; 7.3818x over previous
import jax
import jax.numpy as jnp
from jax import lax
from jax.experimental import pallas as pl
from jax.experimental.pallas import tpu as pltpu

_B, _A, _C = 32, 20000, 81
_NEG_RATIO = 3
_ALPHA = 1.0
_RB = 4 * _A // 128


def _smooth_l1(x):
    ax = jnp.abs(x)
    return jnp.where(ax < 1.0, 0.5 * x * x, ax - 0.5)


def _sortable_u32(x):
    ub = lax.bitcast_convert_type(x, jnp.uint32)
    neg = (ub >> jnp.uint32(31)) == jnp.uint32(1)
    return jnp.where(neg, ~ub, ub | jnp.uint32(0x80000000))


def _unsortable_f32(u):
    bits = jnp.where(u >= jnp.uint32(0x80000000), u ^ jnp.uint32(0x80000000), ~u)
    return lax.bitcast_convert_type(bits, jnp.float32)


def _body(bi_ref, bt_ref, lab_ref, tgt_ref, tgtb_ref, main_ref, bbox_ref,
          lls_ref, facc_ref):
    i = pl.program_id(0)

    @pl.when(i == 0)
    def _init():
        facc_ref[0] = 0.0
        facc_ref[1] = 0.0
        facc_ref[2] = 0.0

    tgt = tgt_ref[...].reshape(1, _A)
    pos = tgt > 0
    posf = pos.astype(jnp.float32)
    npos = jnp.sum(pos.astype(jnp.int32))
    k = jnp.minimum(_NEG_RATIO * npos, _A)

    lab = lab_ref[0]
    cio = lax.broadcasted_iota(jnp.int32, (_C, _A), 0)
    ll = -jnp.sum(jnp.where(cio == tgt, lab, 0.0), axis=0, keepdims=True)

    sum_pos = jnp.sum(ll * posf)

    def _fast(_):
        return jnp.sum(ll * (1.0 - posf))

    def _slow(_):
        ll2 = jnp.where(pos, 0.0, -ll)
        u = _sortable_u32(ll2)

        def bit_step(t, carry):
            prefix, kk = carry
            sh = 31 - t
            cnt0 = jnp.sum(((u >> sh) == (prefix >> sh)).astype(jnp.int32))
            take0 = kk <= cnt0
            prefix = jnp.where(take0, prefix, prefix | (jnp.uint32(1) << sh))
            kk = jnp.where(take0, kk, kk - cnt0)
            return prefix, kk

        t_u, _kk = lax.fori_loop(0, 32, bit_step, (jnp.uint32(0), k))
        lt = u < t_u
        count_lt = jnp.sum(lt.astype(jnp.int32))
        r = (k - count_lt).astype(jnp.float32)
        t_f = _unsortable_f32(t_u)
        s_lt = jnp.sum(-ll2 * jnp.where(lt, 1.0, 0.0))
        return s_lt + r * (-t_f)

    s_neg = lax.cond(k >= _A, _fast, _slow, None)
    s_neg = jnp.where(k > 0, s_neg, 0.0)

    s = _smooth_l1(bi_ref[0] - bt_ref[0])
    kio_l = lax.broadcasted_iota(jnp.int32, (128, 32), 0)
    kio_m = lax.broadcasted_iota(jnp.int32, (128, 32), 1)
    kmat = ((kio_l // 4) == kio_m).astype(jnp.float32)
    s4 = lax.dot_general(s, kmat, (((1,), (0,)), ((), ())),
                         preferred_element_type=jnp.float32)
    posf2 = (tgtb_ref[0] > 0).astype(jnp.float32)

    facc_ref[0] += jnp.sum(s4 * posf2)
    facc_ref[1] += sum_pos + s_neg
    facc_ref[2] += npos.astype(jnp.float32)

    @pl.when(i == _B - 1)
    def _fin():
        np_t = facc_ref[2]
        bb = facc_ref[0] / np_t
        lls = facc_ref[1] / np_t
        bbox_ref[...] = jnp.reshape(bb, (1, 1))
        lls_ref[...] = jnp.reshape(lls, (1, 1))
        main_ref[...] = jnp.reshape(bb + _ALPHA * lls, (1, 1))


def kernel(bbox_input, bbox_target, label_input, label_target):
    bi = bbox_input.reshape(_B, _RB, 128)
    bt = bbox_target.reshape(_B, _RB, 128)
    tgt = label_target.reshape(_B, 1, _A)
    tgtb = label_target.reshape(_B, _RB, 32)

    out_shape = [jax.ShapeDtypeStruct((1, 1), jnp.float32)] * 3
    main, bbox, lls = pl.pallas_call(
        _body,
        grid=(_B,),
        in_specs=[
            pl.BlockSpec((1, _RB, 128), lambda i: (i, 0, 0)),
            pl.BlockSpec((1, _RB, 128), lambda i: (i, 0, 0)),
            pl.BlockSpec((1, _C, _A), lambda i: (i, 0, 0)),
            pl.BlockSpec((1, 1, _A), lambda i: (i, 0, 0)),
            pl.BlockSpec((1, _RB, 32), lambda i: (i, 0, 0)),
        ],
        out_specs=[
            pl.BlockSpec((1, 1), lambda i: (0, 0)),
            pl.BlockSpec((1, 1), lambda i: (0, 0)),
            pl.BlockSpec((1, 1), lambda i: (0, 0)),
        ],
        out_shape=out_shape,
        scratch_shapes=[pltpu.SMEM((3,), jnp.float32)],
        compiler_params=pltpu.CompilerParams(
            dimension_semantics=("arbitrary",),
        ),
    )(bi, bt, label_input, tgt, tgtb)
    return (main[0, 0], bbox[0, 0], lls[0, 0])


# speedup vs baseline: 26.0282x; 3.5260x over previous
import jax
import jax.numpy as jnp
from jax import lax
from jax.experimental import pallas as pl
from jax.experimental.pallas import tpu as pltpu

_B, _A, _C = 32, 20000, 81
_NEG_RATIO = 3
_ALPHA = 1.0
_CBLK = 3
_NSTEP = _C // _CBLK


def _smooth_l1(x):
    ax = jnp.abs(x)
    return jnp.where(ax < 1.0, 0.5 * x * x, ax - 0.5)


def _sortable_u32(x):
    ub = lax.bitcast_convert_type(x, jnp.uint32)
    neg = (ub >> jnp.uint32(31)) == jnp.uint32(1)
    return jnp.where(neg, ~ub, ub | jnp.uint32(0x80000000))


def _unsortable_f32(u):
    bits = jnp.where(u >= jnp.uint32(0x80000000), u ^ jnp.uint32(0x80000000), ~u)
    return lax.bitcast_convert_type(bits, jnp.float32)


def _bbox_body(bi_ref, bt_ref, tgt_ref, out_ref, acc_ref):
    i = pl.program_id(0)

    @pl.when(i == 0)
    def _init():
        acc_ref[0] = 0.0

    s = _smooth_l1(bi_ref[0] - bt_ref[0])
    s4 = jnp.sum(s, axis=0, keepdims=True)
    posf = (tgt_ref[0] > 0).astype(jnp.float32)
    acc_ref[0] += jnp.sum(s4 * posf)

    @pl.when(i == _B - 1)
    def _fin():
        out_ref[...] = jnp.reshape(acc_ref[0], (1, 1))


def _label_body(lab_ref, tgt_ref, bsum_ref, main_ref, bbox_ref, lls_ref,
                acc_ref):
    j = pl.program_id(0)

    @pl.when(j == 0)
    def _init():
        acc_ref[...] = jnp.zeros((_B, _A), jnp.float32)

    tgt = tgt_ref[...]
    lab = lab_ref[...].reshape(_CBLK, _B, _A)
    c0 = j * _CBLK
    part = acc_ref[...]
    for r in range(_CBLK):
        part = part + jnp.where(tgt == (c0 + r), lab[r], 0.0)
    acc_ref[...] = part

    @pl.when(j == _NSTEP - 1)
    def _fin():
        pos = tgt > 0
        posf = pos.astype(jnp.float32)
        npos = jnp.sum(posf, axis=1, keepdims=True)
        nposi = npos.astype(jnp.int32)
        k = jnp.minimum(_NEG_RATIO * nposi, _A)

        ll = -acc_ref[...]
        sum_pos = jnp.sum(ll * posf, axis=1, keepdims=True)
        negsum = jnp.sum(ll * (1.0 - posf), axis=1, keepdims=True)

        need_slow = jnp.any(k < _A)

        def _slow(_):
            ll2 = jnp.where(pos, 0.0, -ll)
            u = _sortable_u32(ll2)

            def bit_step(t, carry):
                prefix, kk = carry
                sh = 31 - t
                eq = (u >> sh) == (prefix >> sh)
                cnt0 = jnp.sum(eq.astype(jnp.int32), axis=1, keepdims=True)
                take0 = kk <= cnt0
                prefix = jnp.where(take0, prefix,
                                   prefix | (jnp.uint32(1) << sh))
                kk = jnp.where(take0, kk, kk - cnt0)
                return prefix, kk

            t_u, _kk = lax.fori_loop(
                0, 32, bit_step,
                (jnp.zeros((_B, 1), jnp.uint32), k))
            lt = u < t_u
            count_lt = jnp.sum(lt.astype(jnp.int32), axis=1, keepdims=True)
            r = (k - count_lt).astype(jnp.float32)
            t_f = _unsortable_f32(t_u)
            s_lt = jnp.sum(-ll2 * jnp.where(lt, 1.0, 0.0), axis=1,
                           keepdims=True)
            return s_lt + r * (-t_f)

        slow_rows = lax.cond(need_slow, _slow,
                             lambda _: jnp.zeros((_B, 1), jnp.float32), None)
        s_neg = jnp.where(k >= _A, negsum,
                          jnp.where(k > 0, slow_rows, 0.0))

        np_t = jnp.sum(npos)
        lls = jnp.sum(sum_pos + s_neg) / np_t
        bb = bsum_ref[0, 0] / np_t
        bbox_ref[...] = jnp.reshape(bb, (1, 1))
        lls_ref[...] = jnp.reshape(lls, (1, 1))
        main_ref[...] = jnp.reshape(bb + _ALPHA * lls, (1, 1))


def kernel(bbox_input, bbox_target, label_input, label_target):
    bi = bbox_input.transpose(0, 2, 1)
    bt = bbox_target.transpose(0, 2, 1)
    lab2 = label_input.transpose(1, 0, 2).reshape(_C * _B, _A)
    tgt3 = label_target.reshape(_B, 1, _A)

    bsum = pl.pallas_call(
        _bbox_body,
        grid=(_B,),
        in_specs=[
            pl.BlockSpec((1, 4, _A), lambda i: (i, 0, 0)),
            pl.BlockSpec((1, 4, _A), lambda i: (i, 0, 0)),
            pl.BlockSpec((1, 1, _A), lambda i: (i, 0, 0)),
        ],
        out_specs=pl.BlockSpec((1, 1), lambda i: (0, 0)),
        out_shape=jax.ShapeDtypeStruct((1, 1), jnp.float32),
        scratch_shapes=[pltpu.SMEM((1,), jnp.float32)],
        compiler_params=pltpu.CompilerParams(
            dimension_semantics=("arbitrary",),
        ),
    )(bi, bt, tgt3)

    out_shape = [jax.ShapeDtypeStruct((1, 1), jnp.float32)] * 3
    main, bbox, lls = pl.pallas_call(
        _label_body,
        grid=(_NSTEP,),
        in_specs=[
            pl.BlockSpec((_CBLK * _B, _A), lambda j: (j, 0)),
            pl.BlockSpec((_B, _A), lambda j: (0, 0)),
            pl.BlockSpec((1, 1), lambda j: (0, 0)),
        ],
        out_specs=[
            pl.BlockSpec((1, 1), lambda j: (0, 0)),
            pl.BlockSpec((1, 1), lambda j: (0, 0)),
            pl.BlockSpec((1, 1), lambda j: (0, 0)),
        ],
        out_shape=out_shape,
        scratch_shapes=[pltpu.VMEM((_B, _A), jnp.float32)],
        compiler_params=pltpu.CompilerParams(
            dimension_semantics=("arbitrary",),
        ),
    )(lab2, label_target, bsum)
    return (main[0, 0], bbox[0, 0], lls[0, 0])


# speedup vs baseline: 30.6199x; 1.1764x over previous
import jax
import jax.numpy as jnp
from jax import lax
from jax.experimental import pallas as pl
from jax.experimental.pallas import tpu as pltpu

_B, _A, _C = 32, 20000, 81
_NEG_RATIO = 3
_ALPHA = 1.0
_CBLK = 3
_NSTEP = _C // _CBLK
_RBLK = 8


def _smooth_l1(x):
    ax = jnp.abs(x)
    return jnp.where(ax < 1.0, 0.5 * x * x, ax - 0.5)


def _sortable_u32(x):
    ub = lax.bitcast_convert_type(x, jnp.uint32)
    neg = (ub >> jnp.uint32(31)) == jnp.uint32(1)
    return jnp.where(neg, ~ub, ub | jnp.uint32(0x80000000))


def _unsortable_f32(u):
    bits = jnp.where(u >= jnp.uint32(0x80000000), u ^ jnp.uint32(0x80000000), ~u)
    return lax.bitcast_convert_type(bits, jnp.float32)


def _bbox_body(bi_ref, bt_ref, tgt_ref, out_ref, acc_ref):
    i = pl.program_id(0)

    @pl.when(i == 0)
    def _init():
        acc_ref[0] = 0.0

    s = _smooth_l1(bi_ref[...] - bt_ref[...])
    s4 = jnp.sum(s, axis=1)
    posf = (tgt_ref[...] > 0).astype(jnp.float32)
    acc_ref[0] += jnp.sum(s4 * posf)

    @pl.when(i == _B // _RBLK - 1)
    def _fin():
        out_ref[...] = jnp.reshape(acc_ref[0], (1, 1))


def _label_body(lab_ref, tgt_ref, bsum_ref, main_ref, bbox_ref, lls_ref,
                acc_ref):
    j = pl.program_id(0)

    @pl.when(j == 0)
    def _init():
        acc_ref[...] = jnp.zeros((_B, _A), jnp.float32)

    tgt = tgt_ref[...]
    lab = lab_ref[...].reshape(_CBLK, _B, _A)
    c0 = j * _CBLK
    part = acc_ref[...]
    for r in range(_CBLK):
        part = part + jnp.where(tgt == (c0 + r), lab[r], 0.0)
    acc_ref[...] = part

    @pl.when(j == _NSTEP - 1)
    def _fin():
        pos = tgt > 0
        posf = pos.astype(jnp.float32)
        npos = jnp.sum(posf, axis=1, keepdims=True)
        nposi = npos.astype(jnp.int32)
        k = jnp.minimum(_NEG_RATIO * nposi, _A)

        ll = -acc_ref[...]
        sum_pos = jnp.sum(ll * posf, axis=1, keepdims=True)
        negsum = jnp.sum(ll * (1.0 - posf), axis=1, keepdims=True)

        need_slow = jnp.any(k < _A)

        def _slow(_):
            ll2 = jnp.where(pos, 0.0, -ll)
            u = _sortable_u32(ll2)

            def bit_step(t, carry):
                prefix, kk = carry
                sh = 31 - t
                eq = (u >> sh) == (prefix >> sh)
                cnt0 = jnp.sum(eq.astype(jnp.int32), axis=1, keepdims=True)
                take0 = kk <= cnt0
                prefix = jnp.where(take0, prefix,
                                   prefix | (jnp.uint32(1) << sh))
                kk = jnp.where(take0, kk, kk - cnt0)
                return prefix, kk

            t_u, _kk = lax.fori_loop(
                0, 32, bit_step,
                (jnp.zeros((_B, 1), jnp.uint32), k))
            lt = u < t_u
            count_lt = jnp.sum(lt.astype(jnp.int32), axis=1, keepdims=True)
            r = (k - count_lt).astype(jnp.float32)
            t_f = _unsortable_f32(t_u)
            s_lt = jnp.sum(-ll2 * jnp.where(lt, 1.0, 0.0), axis=1,
                           keepdims=True)
            return s_lt + r * (-t_f)

        slow_rows = lax.cond(need_slow, _slow,
                             lambda _: jnp.zeros((_B, 1), jnp.float32), None)
        s_neg = jnp.where(k >= _A, negsum,
                          jnp.where(k > 0, slow_rows, 0.0))

        np_t = jnp.sum(npos)
        lls = jnp.sum(sum_pos + s_neg) / np_t
        bb = bsum_ref[0, 0] / np_t
        bbox_ref[...] = jnp.reshape(bb, (1, 1))
        lls_ref[...] = jnp.reshape(lls, (1, 1))
        main_ref[...] = jnp.reshape(bb + _ALPHA * lls, (1, 1))


def kernel(bbox_input, bbox_target, label_input, label_target):
    bi = bbox_input.transpose(0, 2, 1)
    bt = bbox_target.transpose(0, 2, 1)
    lab2 = label_input.transpose(1, 0, 2).reshape(_C * _B, _A)

    bsum = pl.pallas_call(
        _bbox_body,
        grid=(_B // _RBLK,),
        in_specs=[
            pl.BlockSpec((_RBLK, 4, _A), lambda i: (i, 0, 0)),
            pl.BlockSpec((_RBLK, 4, _A), lambda i: (i, 0, 0)),
            pl.BlockSpec((_RBLK, _A), lambda i: (i, 0)),
        ],
        out_specs=pl.BlockSpec((1, 1), lambda i: (0, 0)),
        out_shape=jax.ShapeDtypeStruct((1, 1), jnp.float32),
        scratch_shapes=[pltpu.SMEM((1,), jnp.float32)],
        compiler_params=pltpu.CompilerParams(
            dimension_semantics=("arbitrary",),
        ),
    )(bi, bt, label_target)

    out_shape = [jax.ShapeDtypeStruct((1, 1), jnp.float32)] * 3
    main, bbox, lls = pl.pallas_call(
        _label_body,
        grid=(_NSTEP,),
        in_specs=[
            pl.BlockSpec((_CBLK * _B, _A), lambda j: (j, 0)),
            pl.BlockSpec((_B, _A), lambda j: (0, 0)),
            pl.BlockSpec((1, 1), lambda j: (0, 0)),
        ],
        out_specs=[
            pl.BlockSpec((1, 1), lambda j: (0, 0)),
            pl.BlockSpec((1, 1), lambda j: (0, 0)),
            pl.BlockSpec((1, 1), lambda j: (0, 0)),
        ],
        out_shape=out_shape,
        scratch_shapes=[pltpu.VMEM((_B, _A), jnp.float32)],
        compiler_params=pltpu.CompilerParams(
            dimension_semantics=("arbitrary",),
        ),
    )(lab2, label_target, bsum)
    return (main[0, 0], bbox[0, 0], lls[0, 0])
